# baseline (speedup 1.0000x reference)
import functools

import jax
import jax.numpy as jnp
from jax import lax
from jax.experimental import pallas as pl
from jax.experimental.pallas import tpu as pltpu
from jax.experimental.pallas import tpu_sc as plsc

B = 16384
C = 5
E = 64
NC, NS = 2, 16
NW = NC * NS
PER_W = B // NW
CHUNK = 128
NCH = PER_W // CHUNK
L = 16

_mesh = plsc.VectorSubcoreMesh(core_axis_name="c", subcore_axis_name="s")


@functools.partial(
    pl.kernel,
    out_type=jax.ShapeDtypeStruct((B // CHUNK, C, CHUNK), jnp.float32),
    mesh=_mesh,
    scratch_types=[
        pltpu.VMEM((CHUNK,), jnp.int32),
        pltpu.VMEM((C, CHUNK), jnp.int32),
        pltpu.VMEM((CHUNK, E), jnp.float32),
        pltpu.VMEM((CHUNK * C, E), jnp.float32),
        pltpu.VMEM((C, CHUNK), jnp.float32),
        pltpu.SemaphoreType.DMA,
    ],
    compiler_params=pltpu.CompilerParams(
        needs_layout_passes=False, use_tc_tiling_on_sc=False),
)
def _w2v(t_hbm, cidx_hbm, wt_hbm, wc_hbm, out_hbm,
         t_idx_v, c_idx_v, wt_v, wc_v, out_v, sem):
    wid = lax.axis_index("s") * NC + lax.axis_index("c")
    lanes = lax.iota(jnp.int32, L)
    for j in range(NCH):
        b0 = wid * PER_W + j * CHUNK
        n = b0 // CHUNK
        pltpu.sync_copy(t_hbm.at[pl.ds(b0, CHUNK)], t_idx_v)
        pltpu.sync_copy(cidx_hbm.at[n], c_idx_v)
        cps = [pltpu.async_copy(wt_hbm.at[t_idx_v], wt_v, sem)]
        for r in range(C):
            cps.append(pltpu.async_copy(
                wc_hbm.at[c_idx_v.at[r]],
                wc_v.at[pl.ds(r * CHUNK, CHUNK)], sem))
        for cp in cps:
            cp.wait()

        for g in range(CHUNK // L):

            def bbody(i, res, g=g):
                b = g * L + i
                w = [wt_v[b, pl.ds(16 * k, L)] for k in range(E // L)]
                m = lanes == i
                new = []
                for c in range(C):
                    r = b * C + c
                    acc = w[0] * wc_v[r, pl.ds(0, L)]
                    for k in range(1, E // L):
                        acc = acc + w[k] * wc_v[r, pl.ds(16 * k, L)]
                    new.append(jnp.where(m, jnp.sum(acc), res[c]))
                return tuple(new)

            res = lax.fori_loop(
                0, L, bbody,
                tuple(jnp.zeros((L,), jnp.float32) for _ in range(C)))
            for c in range(C):
                out_v[c, pl.ds(g * L, L)] = res[c]

        pltpu.sync_copy(out_v, out_hbm.at[n])


def kernel(target, context, W_target, W_context):
    cidx = context.reshape(-1).reshape(B // CHUNK, C, CHUNK)
    out = _w2v(target, cidx, W_target, W_context)
    return out.transpose(0, 2, 1).reshape(B, C)
